# baseline (speedup 1.0000x reference)
import functools

import jax
import jax.numpy as jnp
from jax import lax
from jax.experimental import pallas as pl
from jax.experimental.pallas import tpu as pltpu
from jax.experimental.pallas import tpu_sc as plsc

N = 10000
E = 320000
D = 128
NC = 2
NS = 16
NW = NC * NS
CH = 64
NCHUNK = 160
EP = NW * NCHUNK * CH
NP = 10112
RPT = NP // NS
DEGW = 16

_HIGH = lax.Precision.HIGHEST

_GDN = lax.GatherDimensionNumbers(
    offset_dims=(), collapsed_slice_dims=(0,), start_index_map=(0,))


def _splat(vec16, o):
  iv = jnp.full((16,), o, jnp.int32)
  return lax.gather(vec16, iv[:, None], _GDN, (1,),
                    mode=lax.GatherScatterMode.PROMISE_IN_BOUNDS)


def _sc_edge_builder():
  mesh = plsc.VectorSubcoreMesh(core_axis_name="c", subcore_axis_name="s")
  out_type = jax.ShapeDtypeStruct((NC, NP, D), jnp.float32)
  scratch = [
      pltpu.VMEM((3, 2 * CH), jnp.int32),
      pltpu.VMEM((3, CH), jnp.int32),
      pltpu.VMEM((3, CH), jnp.float32),
      pltpu.VMEM((2, 2 * CH, D), jnp.float32),
      pltpu.VMEM((D,), jnp.float32),
      pltpu.VMEM_SHARED((NP, D), jnp.float32),
      pltpu.SemaphoreType.DMA,
      pltpu.SemaphoreType.DMA,
      pltpu.SemaphoreType.DMA,
      pltpu.SemaphoreType.DMA,
  ]

  def body(ht, cidx1, dst1, len1, w1c, z128, s_out,
           cidxv, dstv, lenv, rows, w1cv, s_sh,
           sem0, sem1, semi, sems_):
    c = lax.axis_index("c")
    s = lax.axis_index("s")
    wid = c * NS + s
    row0 = s * RPT
    pltpu.sync_copy(z128.at[pl.ds(row0, RPT)], s_sh.at[pl.ds(row0, RPT)])
    pltpu.sync_copy(w1c, w1cv)
    plsc.subcore_barrier()

    w1r = [w1cv[pl.ds(k * 16, 16)] for k in range(D // 16)]

    ebase = wid * (NCHUNK * CH)
    sems = (sem0, sem1)

    def start_idx(jj, m):
      off = ebase + jj * CH
      pltpu.async_copy(cidx1.at[pl.ds(2 * off, 2 * CH)], cidxv.at[m], semi)
      pltpu.async_copy(dst1.at[pl.ds(off, CH)], dstv.at[m], semi)
      pltpu.async_copy(len1.at[pl.ds(off, CH)], lenv.at[m], semi)

    def wait_idx(m):
      pltpu.make_async_copy(cidx1.at[pl.ds(0, 2 * CH)], cidxv.at[m], semi).wait()
      pltpu.make_async_copy(dst1.at[pl.ds(0, CH)], dstv.at[m], semi).wait()
      pltpu.make_async_copy(len1.at[pl.ds(0, CH)], lenv.at[m], semi).wait()

    def start_gather(b, m):
      pltpu.async_copy(ht.at[cidxv.at[m]], rows.at[b], sems[b])

    def wait_gather(b, m):
      pltpu.make_async_copy(ht.at[cidxv.at[m]], rows.at[b], sems[b]).wait()

    def wait_scatter(b, m):
      pltpu.make_async_copy(rows.at[b, pl.ds(0, CH)],
                            s_sh.at[dstv.at[m]], sems_).wait()

    start_idx(0, 0)
    start_idx(1, 1)
    wait_idx(0)
    start_gather(0, 0)

    def outer_body(t, carry):
      for b in range(2):
        j = 2 * t + b
        m = lax.rem(j, 3)
        mn = lax.rem(j + 1, 3)
        mp = lax.rem(j + 2, 3)
        wait_idx(mn)

        @pl.when(j >= 1)
        def _():
          wait_scatter(1 - b, lax.rem(j + 2, 3))

        start_gather(1 - b, mn)
        start_idx(lax.rem(j + 2, NCHUNK), mp)
        wait_gather(b, m)

        @plsc.parallel_loop(0, CH, unroll=4)
        def _edge_loop(e):
          g16 = (e // 16) * 16
          len16 = lenv[m, pl.ds(g16, 16)]
          lspl = _splat(len16, e - g16)
          for k in range(D // 16):
            sl = pl.ds(k * 16, 16)
            p = rows[b, e, sl] + rows[b, CH + e, sl] + lspl * w1r[k]
            rows[b, e, sl] = p / (1.0 + jnp.exp(-p))

        pltpu.async_copy(rows.at[b, pl.ds(0, CH)], s_sh.at[dstv.at[m]],
                         sems_, add=True)
      return carry

    lax.fori_loop(0, NCHUNK // 2, outer_body, 0)
    wait_scatter(1, (NCHUNK - 1) % 3)
    wait_gather(0, NCHUNK % 3)
    wait_idx((NCHUNK + 1) % 3)
    plsc.subcore_barrier()
    pltpu.sync_copy(s_sh.at[pl.ds(row0, RPT)], s_out.at[c, pl.ds(row0, RPT)])

  return pl.kernel(body, out_type=out_type, mesh=mesh, scratch_types=scratch)


_sc_edge = _sc_edge_builder()


CHD = 128
NCHUNKD = EP // NW // CHD


def _sc_deg_builder():
  mesh = plsc.VectorSubcoreMesh(core_axis_name="c", subcore_axis_name="s")
  out_type = jax.ShapeDtypeStruct((NC, NP, DEGW), jnp.float32)
  scratch = [
      pltpu.VMEM((2, CHD), jnp.int32),
      pltpu.VMEM((CHD, DEGW), jnp.float32),
      pltpu.VMEM_SHARED((NP, DEGW), jnp.float32),
      pltpu.SemaphoreType.DMA,
      pltpu.SemaphoreType.DMA,
  ]

  def body(dst1, ones_h, z16, deg_out, dstv, onesv, deg_sh, semi, sems_):
    c = lax.axis_index("c")
    s = lax.axis_index("s")
    wid = c * NS + s
    row0 = s * RPT
    pltpu.sync_copy(z16.at[pl.ds(row0, RPT)], deg_sh.at[pl.ds(row0, RPT)])
    pltpu.sync_copy(ones_h, onesv)
    plsc.subcore_barrier()
    ebase = wid * (NCHUNKD * CHD)

    def start_idx(jj, b):
      pltpu.async_copy(dst1.at[pl.ds(ebase + jj * CHD, CHD)], dstv.at[b], semi)

    def wait_idx(b):
      pltpu.make_async_copy(dst1.at[pl.ds(0, CHD)], dstv.at[b], semi).wait()

    def wait_scatter(b):
      pltpu.make_async_copy(onesv, deg_sh.at[dstv.at[b]], sems_).wait()

    start_idx(0, 0)

    def outer_body(t, carry):
      for b in range(2):
        j = 2 * t + b

        @pl.when(j >= 1)
        def _():
          wait_scatter(1 - b)

        start_idx(lax.rem(j + 1, NCHUNKD), 1 - b)
        wait_idx(b)
        pltpu.async_copy(onesv, deg_sh.at[dstv.at[b]], sems_, add=True)
      return carry

    lax.fori_loop(0, NCHUNKD // 2, outer_body, 0)
    wait_scatter(1)
    wait_idx(0)
    plsc.subcore_barrier()
    pltpu.sync_copy(deg_sh.at[pl.ds(row0, RPT)],
                    deg_out.at[c, pl.ds(row0, RPT)])

  return pl.kernel(body, out_type=out_type, mesh=mesh, scratch_types=scratch)


_sc_deg = _sc_deg_builder()


R = 2000


def _rows(i):
  return pl.BlockSpec((R, D), lambda g: (g, 0))


def _wmat():
  return pl.BlockSpec((D, D), lambda g: (0, 0))


def _brow():
  return pl.BlockSpec((1, D), lambda g: (0, 0))


def _degs():
  return pl.BlockSpec((R, DEGW), lambda g: (g, 0))


def _prep_body(h_ref, wa_ref, wb_ref, b1_ref, ha_ref, hb_ref):
  x = h_ref[...]
  ha_ref[...] = jnp.dot(x, wa_ref[...], precision=_HIGH)
  hb_ref[...] = jnp.dot(x, wb_ref[...], precision=_HIGH) + b1_ref[...]


_prep = pl.pallas_call(
    _prep_body,
    grid=(N // R,),
    in_specs=[_rows(0), _wmat(), _wmat(), _brow()],
    out_specs=[_rows(0), _rows(0)],
    out_shape=[jax.ShapeDtypeStruct((N, D), jnp.float32)] * 2,
)


def _mid_body(h_ref, sa_ref, sb_ref, da_ref, db_ref, w2_ref, b2_ref,
              wa_ref, wb_ref, nb1_ref, h1_ref, ha_ref, hb_ref):
  ssum = sa_ref[...] + sb_ref[...]
  deg = da_ref[:, 0:1] + db_ref[:, 0:1]
  h1 = h_ref[...] + jnp.dot(ssum, w2_ref[...], precision=_HIGH) + deg * b2_ref[...]
  h1_ref[...] = h1
  ha_ref[...] = jnp.dot(h1, wa_ref[...], precision=_HIGH)
  hb_ref[...] = jnp.dot(h1, wb_ref[...], precision=_HIGH) + nb1_ref[...]


_mid = pl.pallas_call(
    _mid_body,
    grid=(N // R,),
    in_specs=[_rows(0), _rows(0), _rows(0), _degs(), _degs(),
              _wmat(), _brow(), _wmat(), _wmat(), _brow()],
    out_specs=[_rows(0), _rows(0), _rows(0)],
    out_shape=[jax.ShapeDtypeStruct((N, D), jnp.float32)] * 3,
)


def _fin_body(h1_ref, sa_ref, sb_ref, da_ref, db_ref, w2_ref, b2_ref,
              nw1_ref, nb1_ref, nw2_ref, nb2_ref, o_ref):
  ssum = sa_ref[...] + sb_ref[...]
  deg = da_ref[:, 0:1] + db_ref[:, 0:1]
  h2 = h1_ref[...] + jnp.dot(ssum, w2_ref[...], precision=_HIGH) + deg * b2_ref[...]
  z = jnp.dot(h2, nw1_ref[...], precision=_HIGH) + nb1_ref[...]
  z = z / (1.0 + jnp.exp(-z))
  o_ref[...] = h2 + jnp.dot(z, nw2_ref[...], precision=_HIGH) + nb2_ref[...]


_fin = pl.pallas_call(
    _fin_body,
    grid=(N // R,),
    in_specs=[_rows(0), _rows(0), _rows(0), _degs(), _degs(),
              _wmat(), _brow(), _wmat(), _brow(), _wmat(), _brow()],
    out_specs=_rows(0),
    out_shape=jax.ShapeDtypeStruct((N, D), jnp.float32),
)


@jax.jit
def kernel(h, edge_index, edge_len,
           mp0_W1, mp0_b1, mp0_W2, mp0_b2,
           mp1_W1, mp1_b1, mp1_W2, mp1_b2,
           node_W1, node_b1, node_W2, node_b2):
  pad = EP - E
  src1 = jnp.concatenate(
      [edge_index[0].astype(jnp.int32), jnp.zeros((pad,), jnp.int32)])
  dst1 = jnp.concatenate(
      [edge_index[1].astype(jnp.int32), jnp.full((pad,), N, jnp.int32)])
  len1 = jnp.concatenate(
      [edge_len.astype(jnp.float32), jnp.zeros((pad,), jnp.float32)])
  z128 = jnp.zeros((NP, D), jnp.float32)
  z16 = jnp.zeros((NP, DEGW), jnp.float32)
  ones_h = jnp.zeros((CHD, DEGW), jnp.float32).at[:, 0].set(1.0)

  sr = src1.reshape(-1, CH)
  dr = jnp.where(dst1 < N, dst1 + N, 0).reshape(-1, CH)
  cidx1 = jnp.stack([sr, dr], axis=1).reshape(-1)

  ha0, hb0 = _prep(h, mp0_W1[:D], mp0_W1[D:2 * D], mp0_b1.reshape(1, D))
  ht0 = jnp.concatenate([ha0, hb0])
  s0 = _sc_edge(ht0, cidx1, dst1, len1, mp0_W1[2 * D], z128)
  deg16 = _sc_deg(dst1, ones_h, z16)
  h1, ha1, hb1 = _mid(h, s0[0], s0[1], deg16[0], deg16[1],
                      mp0_W2, mp0_b2.reshape(1, D),
                      mp1_W1[:D], mp1_W1[D:2 * D], mp1_b1.reshape(1, D))
  ht1 = jnp.concatenate([ha1, hb1])
  s1 = _sc_edge(ht1, cidx1, dst1, len1, mp1_W1[2 * D], z128)
  out = _fin(h1, s1[0], s1[1], deg16[0], deg16[1],
             mp1_W2, mp1_b2.reshape(1, D),
             node_W1, node_b1.reshape(1, D),
             node_W2, node_b2.reshape(1, D))
  return out


# speedup vs baseline: 1.5612x; 1.5612x over previous
import functools

import jax
import jax.numpy as jnp
from jax import lax
from jax.experimental import pallas as pl
from jax.experimental.pallas import tpu as pltpu
from jax.experimental.pallas import tpu_sc as plsc

N = 10000
E = 320000
D = 128
NC = 2
NS = 16
NW = NC * NS
CH = 64
NCHUNK = 160
EP = NW * NCHUNK * CH
NP = 10112
RPT = NP // NS
DEGW = 16

_HIGH = lax.Precision.HIGHEST

_GDN = lax.GatherDimensionNumbers(
    offset_dims=(), collapsed_slice_dims=(0,), start_index_map=(0,))


def _splat(vec16, o):
  iv = jnp.full((16,), o, jnp.int32)
  return lax.gather(vec16, iv[:, None], _GDN, (1,),
                    mode=lax.GatherScatterMode.PROMISE_IN_BOUNDS)


def _sc_edge_builder():
  mesh = plsc.VectorSubcoreMesh(core_axis_name="c", subcore_axis_name="s")
  out_type = jax.ShapeDtypeStruct((NC, NP, D), jnp.float32)
  scratch = [
      pltpu.VMEM((3, CH), jnp.int32),
      pltpu.VMEM((3, CH), jnp.int32),
      pltpu.VMEM((3, CH), jnp.float32),
      pltpu.VMEM((2, CH, D), jnp.float32),
      pltpu.VMEM((2, CH, D), jnp.float32),
      pltpu.VMEM((D,), jnp.float32),
      pltpu.VMEM_SHARED((NP, D), jnp.float32),
      pltpu.SemaphoreType.DMA,
      pltpu.SemaphoreType.DMA,
      pltpu.SemaphoreType.DMA,
      pltpu.SemaphoreType.DMA,
  ]

  def body(ha, hb, src1, dst1, len1, w1c, z128, s_out,
           srcv, dstv, lenv, rowsa, rowsb, w1cv, s_sh,
           sem0, sem1, semi, sems_):
    c = lax.axis_index("c")
    s = lax.axis_index("s")
    wid = c * NS + s
    row0 = s * RPT
    pltpu.sync_copy(z128.at[pl.ds(row0, RPT)], s_sh.at[pl.ds(row0, RPT)])
    pltpu.sync_copy(w1c, w1cv)
    plsc.subcore_barrier()

    w1r = [w1cv[pl.ds(k * 16, 16)] for k in range(D // 16)]

    ebase = wid * (NCHUNK * CH)
    sems = (sem0, sem1)

    def start_idx(jj, m):
      off = ebase + jj * CH
      pltpu.async_copy(src1.at[pl.ds(off, CH)], srcv.at[m], semi)
      pltpu.async_copy(dst1.at[pl.ds(off, CH)], dstv.at[m], semi)
      pltpu.async_copy(len1.at[pl.ds(off, CH)], lenv.at[m], semi)

    def wait_idx(m):
      pltpu.make_async_copy(src1.at[pl.ds(0, CH)], srcv.at[m], semi).wait()
      pltpu.make_async_copy(dst1.at[pl.ds(0, CH)], dstv.at[m], semi).wait()
      pltpu.make_async_copy(len1.at[pl.ds(0, CH)], lenv.at[m], semi).wait()

    HH = CH // 2

    def start_gather(b, m):
      for hof in (0, HH):
        sl = pl.ds(hof, HH)
        pltpu.async_copy(ha.at[srcv.at[m, sl]], rowsa.at[b, sl], sems[b])
        pltpu.async_copy(hb.at[dstv.at[m, sl]], rowsb.at[b, sl], sems[b])

    def wait_gather(b, m):
      for hof in (0, HH):
        sl = pl.ds(hof, HH)
        pltpu.make_async_copy(ha.at[srcv.at[m, sl]], rowsa.at[b, sl],
                              sems[b]).wait()
        pltpu.make_async_copy(hb.at[dstv.at[m, sl]], rowsb.at[b, sl],
                              sems[b]).wait()

    def wait_scatter(b, m):
      pltpu.make_async_copy(rowsa.at[b], s_sh.at[dstv.at[m]], sems_).wait()

    start_idx(0, 0)
    start_idx(1, 1)
    wait_idx(0)
    start_gather(0, 0)

    def outer_body(t, carry):
      for b in range(2):
        j = 2 * t + b
        m = lax.rem(j, 3)
        mn = lax.rem(j + 1, 3)
        mp = lax.rem(j + 2, 3)
        wait_idx(mn)

        @pl.when(j >= 1)
        def _():
          wait_scatter(1 - b, lax.rem(j + 2, 3))

        start_gather(1 - b, mn)
        start_idx(lax.rem(j + 2, NCHUNK), mp)
        wait_gather(b, m)

        @plsc.parallel_loop(0, CH, unroll=4)
        def _edge_loop(e):
          g16 = (e // 16) * 16
          len16 = lenv[m, pl.ds(g16, 16)]
          lspl = _splat(len16, e - g16)
          for k in range(D // 16):
            sl = pl.ds(k * 16, 16)
            p = rowsa[b, e, sl] + rowsb[b, e, sl] + lspl * w1r[k]
            rowsa[b, e, sl] = p / (1.0 + jnp.exp(-p))

        pltpu.async_copy(rowsa.at[b], s_sh.at[dstv.at[m]], sems_, add=True)
      return carry

    lax.fori_loop(0, NCHUNK // 2, outer_body, 0)
    wait_scatter(1, (NCHUNK - 1) % 3)
    wait_gather(0, NCHUNK % 3)
    wait_idx((NCHUNK + 1) % 3)
    plsc.subcore_barrier()
    pltpu.sync_copy(s_sh.at[pl.ds(row0, RPT)], s_out.at[c, pl.ds(row0, RPT)])

  return pl.kernel(body, out_type=out_type, mesh=mesh, scratch_types=scratch)


_sc_edge = _sc_edge_builder()


CHD = 128
NCHUNKD = EP // NW // CHD


def _sc_deg_builder():
  mesh = plsc.VectorSubcoreMesh(core_axis_name="c", subcore_axis_name="s")
  out_type = jax.ShapeDtypeStruct((NC, NP, DEGW), jnp.float32)
  scratch = [
      pltpu.VMEM((2, CHD), jnp.int32),
      pltpu.VMEM((CHD, DEGW), jnp.float32),
      pltpu.VMEM_SHARED((NP, DEGW), jnp.float32),
      pltpu.SemaphoreType.DMA,
      pltpu.SemaphoreType.DMA,
  ]

  def body(dst1, ones_h, z16, deg_out, dstv, onesv, deg_sh, semi, sems_):
    c = lax.axis_index("c")
    s = lax.axis_index("s")
    wid = c * NS + s
    row0 = s * RPT
    pltpu.sync_copy(z16.at[pl.ds(row0, RPT)], deg_sh.at[pl.ds(row0, RPT)])
    pltpu.sync_copy(ones_h, onesv)
    plsc.subcore_barrier()
    ebase = wid * (NCHUNKD * CHD)

    def start_idx(jj, b):
      pltpu.async_copy(dst1.at[pl.ds(ebase + jj * CHD, CHD)], dstv.at[b], semi)

    def wait_idx(b):
      pltpu.make_async_copy(dst1.at[pl.ds(0, CHD)], dstv.at[b], semi).wait()

    def wait_scatter(b):
      pltpu.make_async_copy(onesv, deg_sh.at[dstv.at[b]], sems_).wait()

    start_idx(0, 0)

    def outer_body(t, carry):
      for b in range(2):
        j = 2 * t + b

        @pl.when(j >= 1)
        def _():
          wait_scatter(1 - b)

        start_idx(lax.rem(j + 1, NCHUNKD), 1 - b)
        wait_idx(b)
        pltpu.async_copy(onesv, deg_sh.at[dstv.at[b]], sems_, add=True)
      return carry

    lax.fori_loop(0, NCHUNKD // 2, outer_body, 0)
    wait_scatter(1)
    wait_idx(0)
    plsc.subcore_barrier()
    pltpu.sync_copy(deg_sh.at[pl.ds(row0, RPT)],
                    deg_out.at[c, pl.ds(row0, RPT)])

  return pl.kernel(body, out_type=out_type, mesh=mesh, scratch_types=scratch)


_sc_deg = _sc_deg_builder()


R = 2000


def _rows(i):
  return pl.BlockSpec((R, D), lambda g: (g, 0))


def _wmat():
  return pl.BlockSpec((D, D), lambda g: (0, 0))


def _brow():
  return pl.BlockSpec((1, D), lambda g: (0, 0))


def _degs():
  return pl.BlockSpec((R, DEGW), lambda g: (g, 0))


def _prep_body(h_ref, wa_ref, wb_ref, b1_ref, ha_ref, hb_ref):
  x = h_ref[...]
  ha_ref[...] = jnp.dot(x, wa_ref[...], precision=_HIGH)
  hb_ref[...] = jnp.dot(x, wb_ref[...], precision=_HIGH) + b1_ref[...]


_prep = pl.pallas_call(
    _prep_body,
    grid=(N // R,),
    in_specs=[_rows(0), _wmat(), _wmat(), _brow()],
    out_specs=[_rows(0), _rows(0)],
    out_shape=[jax.ShapeDtypeStruct((N, D), jnp.float32)] * 2,
)


def _mid_body(h_ref, sa_ref, sb_ref, da_ref, db_ref, w2_ref, b2_ref,
              wa_ref, wb_ref, nb1_ref, h1_ref, ha_ref, hb_ref):
  ssum = sa_ref[...] + sb_ref[...]
  deg = da_ref[:, 0:1] + db_ref[:, 0:1]
  h1 = h_ref[...] + jnp.dot(ssum, w2_ref[...], precision=_HIGH) + deg * b2_ref[...]
  h1_ref[...] = h1
  ha_ref[...] = jnp.dot(h1, wa_ref[...], precision=_HIGH)
  hb_ref[...] = jnp.dot(h1, wb_ref[...], precision=_HIGH) + nb1_ref[...]


_mid = pl.pallas_call(
    _mid_body,
    grid=(N // R,),
    in_specs=[_rows(0), _rows(0), _rows(0), _degs(), _degs(),
              _wmat(), _brow(), _wmat(), _wmat(), _brow()],
    out_specs=[_rows(0), _rows(0), _rows(0)],
    out_shape=[jax.ShapeDtypeStruct((N, D), jnp.float32)] * 3,
)


def _fin_body(h1_ref, sa_ref, sb_ref, da_ref, db_ref, w2_ref, b2_ref,
              nw1_ref, nb1_ref, nw2_ref, nb2_ref, o_ref):
  ssum = sa_ref[...] + sb_ref[...]
  deg = da_ref[:, 0:1] + db_ref[:, 0:1]
  h2 = h1_ref[...] + jnp.dot(ssum, w2_ref[...], precision=_HIGH) + deg * b2_ref[...]
  z = jnp.dot(h2, nw1_ref[...], precision=_HIGH) + nb1_ref[...]
  z = z / (1.0 + jnp.exp(-z))
  o_ref[...] = h2 + jnp.dot(z, nw2_ref[...], precision=_HIGH) + nb2_ref[...]


_fin = pl.pallas_call(
    _fin_body,
    grid=(N // R,),
    in_specs=[_rows(0), _rows(0), _rows(0), _degs(), _degs(),
              _wmat(), _brow(), _wmat(), _brow(), _wmat(), _brow()],
    out_specs=_rows(0),
    out_shape=jax.ShapeDtypeStruct((N, D), jnp.float32),
)


@jax.jit
def kernel(h, edge_index, edge_len,
           mp0_W1, mp0_b1, mp0_W2, mp0_b2,
           mp1_W1, mp1_b1, mp1_W2, mp1_b2,
           node_W1, node_b1, node_W2, node_b2):
  pad = EP - E
  src1 = jnp.concatenate(
      [edge_index[0].astype(jnp.int32), jnp.zeros((pad,), jnp.int32)])
  dst1 = jnp.concatenate(
      [edge_index[1].astype(jnp.int32), jnp.full((pad,), N, jnp.int32)])
  len1 = jnp.concatenate(
      [edge_len.astype(jnp.float32), jnp.zeros((pad,), jnp.float32)])
  z128 = jnp.zeros((NP, D), jnp.float32)
  z16 = jnp.zeros((NP, DEGW), jnp.float32)
  ones_h = jnp.zeros((CHD, DEGW), jnp.float32).at[:, 0].set(1.0)

  ha0, hb0 = _prep(h, mp0_W1[:D], mp0_W1[D:2 * D], mp0_b1.reshape(1, D))
  s0 = _sc_edge(ha0, hb0, src1, dst1, len1, mp0_W1[2 * D], z128)
  deg16 = _sc_deg(dst1, ones_h, z16)
  h1, ha1, hb1 = _mid(h, s0[0], s0[1], deg16[0], deg16[1],
                      mp0_W2, mp0_b2.reshape(1, D),
                      mp1_W1[:D], mp1_W1[D:2 * D], mp1_b1.reshape(1, D))
  s1 = _sc_edge(ha1, hb1, src1, dst1, len1, mp1_W1[2 * D], z128)
  out = _fin(h1, s1[0], s1[1], deg16[0], deg16[1],
             mp1_W2, mp1_b2.reshape(1, D),
             node_W1, node_b1.reshape(1, D),
             node_W2, node_b2.reshape(1, D))
  return out


# speedup vs baseline: 1.5811x; 1.0128x over previous
import functools

import jax
import jax.numpy as jnp
from jax import lax
from jax.experimental import pallas as pl
from jax.experimental.pallas import tpu as pltpu
from jax.experimental.pallas import tpu_sc as plsc

N = 10000
E = 320000
D = 128
NC = 2
NS = 16
NW = NC * NS
CH = 64
NCHUNK = 160
EP = NW * NCHUNK * CH
NP = 10112
RPT = NP // NS
DEGW = 8

_HIGH = lax.Precision.HIGHEST

_GDN = lax.GatherDimensionNumbers(
    offset_dims=(), collapsed_slice_dims=(0,), start_index_map=(0,))


def _splat(vec16, o):
  iv = jnp.full((16,), o, jnp.int32)
  return lax.gather(vec16, iv[:, None], _GDN, (1,),
                    mode=lax.GatherScatterMode.PROMISE_IN_BOUNDS)


def _sc_edge_builder():
  mesh = plsc.VectorSubcoreMesh(core_axis_name="c", subcore_axis_name="s")
  out_type = jax.ShapeDtypeStruct((NC, NP, D), jnp.float32)
  scratch = [
      pltpu.VMEM((3, CH), jnp.int32),
      pltpu.VMEM((3, CH), jnp.int32),
      pltpu.VMEM((3, CH), jnp.float32),
      pltpu.VMEM((2, CH, D), jnp.float32),
      pltpu.VMEM((2, CH, D), jnp.float32),
      pltpu.VMEM((D,), jnp.float32),
      pltpu.VMEM_SHARED((NP, D), jnp.float32),
      pltpu.SemaphoreType.DMA,
      pltpu.SemaphoreType.DMA,
      pltpu.SemaphoreType.DMA,
      pltpu.SemaphoreType.DMA,
  ]

  def body(ha, hb, src1, dst1, len1, w1c, z128, s_out,
           srcv, dstv, lenv, rowsa, rowsb, w1cv, s_sh,
           sem0, sem1, semi, sems_):
    c = lax.axis_index("c")
    s = lax.axis_index("s")
    wid = c * NS + s
    row0 = s * RPT
    pltpu.sync_copy(z128.at[pl.ds(row0, RPT)], s_sh.at[pl.ds(row0, RPT)])
    pltpu.sync_copy(w1c, w1cv)
    plsc.subcore_barrier()

    w1r = [w1cv[pl.ds(k * 16, 16)] for k in range(D // 16)]

    ebase = wid * (NCHUNK * CH)
    sems = (sem0, sem1)

    def start_idx(jj, m):
      off = ebase + jj * CH
      pltpu.async_copy(src1.at[pl.ds(off, CH)], srcv.at[m], semi)
      pltpu.async_copy(dst1.at[pl.ds(off, CH)], dstv.at[m], semi)
      pltpu.async_copy(len1.at[pl.ds(off, CH)], lenv.at[m], semi)

    def wait_idx(m):
      pltpu.make_async_copy(src1.at[pl.ds(0, CH)], srcv.at[m], semi).wait()
      pltpu.make_async_copy(dst1.at[pl.ds(0, CH)], dstv.at[m], semi).wait()
      pltpu.make_async_copy(len1.at[pl.ds(0, CH)], lenv.at[m], semi).wait()

    def start_gather(b, m):
      pltpu.async_copy(ha.at[srcv.at[m]], rowsa.at[b], sems[b])
      pltpu.async_copy(hb.at[dstv.at[m]], rowsb.at[b], sems[b])

    def wait_gather(b, m):
      pltpu.make_async_copy(ha.at[srcv.at[m]], rowsa.at[b], sems[b]).wait()
      pltpu.make_async_copy(hb.at[dstv.at[m]], rowsb.at[b], sems[b]).wait()

    def wait_scatter(b, m):
      pltpu.make_async_copy(rowsa.at[b], s_sh.at[dstv.at[m]], sems_).wait()

    start_idx(0, 0)
    start_idx(1, 1)
    wait_idx(0)
    start_gather(0, 0)

    def outer_body(t, carry):
      for b in range(2):
        j = 2 * t + b
        m = lax.rem(j, 3)
        mn = lax.rem(j + 1, 3)
        mp = lax.rem(j + 2, 3)
        wait_idx(mn)

        @pl.when(j >= 1)
        def _():
          wait_scatter(1 - b, lax.rem(j + 2, 3))

        start_gather(1 - b, mn)
        start_idx(lax.rem(j + 2, NCHUNK), mp)
        wait_gather(b, m)

        @plsc.parallel_loop(0, CH, unroll=4)
        def _edge_loop(e):
          g16 = (e // 16) * 16
          len16 = lenv[m, pl.ds(g16, 16)]
          lspl = _splat(len16, e - g16)
          for k in range(D // 16):
            sl = pl.ds(k * 16, 16)
            p = rowsa[b, e, sl] + rowsb[b, e, sl] + lspl * w1r[k]
            rowsa[b, e, sl] = p / (1.0 + jnp.exp(-p))

        pltpu.async_copy(rowsa.at[b], s_sh.at[dstv.at[m]], sems_, add=True)
      return carry

    lax.fori_loop(0, NCHUNK // 2, outer_body, 0)
    wait_scatter(1, (NCHUNK - 1) % 3)
    wait_gather(0, NCHUNK % 3)
    wait_idx((NCHUNK + 1) % 3)
    plsc.subcore_barrier()
    pltpu.sync_copy(s_sh.at[pl.ds(row0, RPT)], s_out.at[c, pl.ds(row0, RPT)])

  return pl.kernel(body, out_type=out_type, mesh=mesh, scratch_types=scratch)


_sc_edge = _sc_edge_builder()


CHD = 128
NCHUNKD = EP // NW // CHD


def _sc_deg_builder():
  mesh = plsc.VectorSubcoreMesh(core_axis_name="c", subcore_axis_name="s")
  out_type = jax.ShapeDtypeStruct((NC, NP, DEGW), jnp.float32)
  scratch = [
      pltpu.VMEM((2, CHD), jnp.int32),
      pltpu.VMEM((CHD, DEGW), jnp.float32),
      pltpu.VMEM_SHARED((NP, DEGW), jnp.float32),
      pltpu.SemaphoreType.DMA,
      pltpu.SemaphoreType.DMA,
  ]

  def body(dst1, ones_h, z16, deg_out, dstv, onesv, deg_sh, semi, sems_):
    c = lax.axis_index("c")
    s = lax.axis_index("s")
    wid = c * NS + s
    row0 = s * RPT
    pltpu.sync_copy(z16.at[pl.ds(row0, RPT)], deg_sh.at[pl.ds(row0, RPT)])
    pltpu.sync_copy(ones_h, onesv)
    plsc.subcore_barrier()
    ebase = wid * (NCHUNKD * CHD)

    def start_idx(jj, b):
      pltpu.async_copy(dst1.at[pl.ds(ebase + jj * CHD, CHD)], dstv.at[b], semi)

    def wait_idx(b):
      pltpu.make_async_copy(dst1.at[pl.ds(0, CHD)], dstv.at[b], semi).wait()

    def wait_scatter(b):
      pltpu.make_async_copy(onesv, deg_sh.at[dstv.at[b]], sems_).wait()

    start_idx(0, 0)

    def outer_body(t, carry):
      for b in range(2):
        j = 2 * t + b

        @pl.when(j >= 1)
        def _():
          wait_scatter(1 - b)

        start_idx(lax.rem(j + 1, NCHUNKD), 1 - b)
        wait_idx(b)
        pltpu.async_copy(onesv, deg_sh.at[dstv.at[b]], sems_, add=True)
      return carry

    lax.fori_loop(0, NCHUNKD // 2, outer_body, 0)
    wait_scatter(1)
    wait_idx(0)
    plsc.subcore_barrier()
    pltpu.sync_copy(deg_sh.at[pl.ds(row0, RPT)],
                    deg_out.at[c, pl.ds(row0, RPT)])

  return pl.kernel(body, out_type=out_type, mesh=mesh, scratch_types=scratch)


_sc_deg = _sc_deg_builder()


R = 2000


def _rows(i):
  return pl.BlockSpec((R, D), lambda g: (g, 0))


def _wmat():
  return pl.BlockSpec((D, D), lambda g: (0, 0))


def _brow():
  return pl.BlockSpec((1, D), lambda g: (0, 0))


def _degs():
  return pl.BlockSpec((R, DEGW), lambda g: (g, 0))


def _prep_body(h_ref, wa_ref, wb_ref, b1_ref, ha_ref, hb_ref):
  x = h_ref[...]
  ha_ref[...] = jnp.dot(x, wa_ref[...], precision=_HIGH)
  hb_ref[...] = jnp.dot(x, wb_ref[...], precision=_HIGH) + b1_ref[...]


_prep = pl.pallas_call(
    _prep_body,
    grid=(N // R,),
    in_specs=[_rows(0), _wmat(), _wmat(), _brow()],
    out_specs=[_rows(0), _rows(0)],
    out_shape=[jax.ShapeDtypeStruct((N, D), jnp.float32)] * 2,
)


def _mid_body(h_ref, sa_ref, sb_ref, da_ref, db_ref, w2_ref, b2_ref,
              wa_ref, wb_ref, nb1_ref, h1_ref, ha_ref, hb_ref):
  ssum = sa_ref[...] + sb_ref[...]
  deg = da_ref[:, 0:1] + db_ref[:, 0:1]
  h1 = h_ref[...] + jnp.dot(ssum, w2_ref[...], precision=_HIGH) + deg * b2_ref[...]
  h1_ref[...] = h1
  ha_ref[...] = jnp.dot(h1, wa_ref[...], precision=_HIGH)
  hb_ref[...] = jnp.dot(h1, wb_ref[...], precision=_HIGH) + nb1_ref[...]


_mid = pl.pallas_call(
    _mid_body,
    grid=(N // R,),
    in_specs=[_rows(0), _rows(0), _rows(0), _degs(), _degs(),
              _wmat(), _brow(), _wmat(), _wmat(), _brow()],
    out_specs=[_rows(0), _rows(0), _rows(0)],
    out_shape=[jax.ShapeDtypeStruct((N, D), jnp.float32)] * 3,
)


def _fin_body(h1_ref, sa_ref, sb_ref, da_ref, db_ref, w2_ref, b2_ref,
              nw1_ref, nb1_ref, nw2_ref, nb2_ref, o_ref):
  ssum = sa_ref[...] + sb_ref[...]
  deg = da_ref[:, 0:1] + db_ref[:, 0:1]
  h2 = h1_ref[...] + jnp.dot(ssum, w2_ref[...], precision=_HIGH) + deg * b2_ref[...]
  z = jnp.dot(h2, nw1_ref[...], precision=_HIGH) + nb1_ref[...]
  z = z / (1.0 + jnp.exp(-z))
  o_ref[...] = h2 + jnp.dot(z, nw2_ref[...], precision=_HIGH) + nb2_ref[...]


_fin = pl.pallas_call(
    _fin_body,
    grid=(N // R,),
    in_specs=[_rows(0), _rows(0), _rows(0), _degs(), _degs(),
              _wmat(), _brow(), _wmat(), _brow(), _wmat(), _brow()],
    out_specs=_rows(0),
    out_shape=jax.ShapeDtypeStruct((N, D), jnp.float32),
)


@jax.jit
def kernel(h, edge_index, edge_len,
           mp0_W1, mp0_b1, mp0_W2, mp0_b2,
           mp1_W1, mp1_b1, mp1_W2, mp1_b2,
           node_W1, node_b1, node_W2, node_b2):
  pad = EP - E
  src1 = jnp.concatenate(
      [edge_index[0].astype(jnp.int32), jnp.zeros((pad,), jnp.int32)])
  dst1 = jnp.concatenate(
      [edge_index[1].astype(jnp.int32), jnp.full((pad,), N, jnp.int32)])
  len1 = jnp.concatenate(
      [edge_len.astype(jnp.float32), jnp.zeros((pad,), jnp.float32)])
  z128 = jnp.zeros((NP, D), jnp.float32)
  z16 = jnp.zeros((NP, DEGW), jnp.float32)
  ones_h = jnp.zeros((CHD, DEGW), jnp.float32).at[:, 0].set(1.0)

  ha0, hb0 = _prep(h, mp0_W1[:D], mp0_W1[D:2 * D], mp0_b1.reshape(1, D))
  s0 = _sc_edge(ha0, hb0, src1, dst1, len1, mp0_W1[2 * D], z128)
  deg16 = _sc_deg(dst1, ones_h, z16)
  h1, ha1, hb1 = _mid(h, s0[0], s0[1], deg16[0], deg16[1],
                      mp0_W2, mp0_b2.reshape(1, D),
                      mp1_W1[:D], mp1_W1[D:2 * D], mp1_b1.reshape(1, D))
  s1 = _sc_edge(ha1, hb1, src1, dst1, len1, mp1_W1[2 * D], z128)
  out = _fin(h1, s1[0], s1[1], deg16[0], deg16[1],
             mp1_W2, mp1_b2.reshape(1, D),
             node_W1, node_b1.reshape(1, D),
             node_W2, node_b2.reshape(1, D))
  return out
